# baseline (speedup 1.0000x reference)
import jax
import jax.numpy as jnp
from jax import lax
from jax.experimental import pallas as pl
from jax.experimental.pallas import tpu as pltpu
from jax.experimental.pallas import tpu_sc as plsc

N = 10000
C = 256
K = 27
KK = K + 1
EPS = 1e-6

NW = 32
NP = 10240
RPW = NP // NW
BN = 2000
CH = 8
NCH = RPW // CH
CHR = CH * KK
CHH = CHR // 2


def _stage_body(x_ref, f_ref, g_ref, b_ref, pb_ref, w_ref, out_ref, h_ref):
    k = pl.program_id(1)

    @pl.when(k == 0)
    def _():
        x = x_ref[...].astype(jnp.float32)
        mean = jnp.mean(x, axis=-1, keepdims=True)
        var = jnp.mean((x - mean) ** 2, axis=-1, keepdims=True)
        y = (x - mean) * lax.rsqrt(var + EPS)
        y = y * g_ref[0, :] + b_ref[0, :]
        h_ref[...] = (y * jax.nn.sigmoid(y)).astype(jnp.bfloat16)

    @pl.when(k < K)
    def _():
        out_ref[0] = jnp.dot(
            h_ref[...], w_ref[jnp.minimum(k, K - 1)],
            preferred_element_type=jnp.float32,
        ).astype(jnp.bfloat16)

    @pl.when(k == K)
    def _():
        out_ref[0] = (f_ref[...].astype(jnp.float32)
                      + pb_ref[0, :]).astype(jnp.bfloat16)


def _stage_matmul(x, resid, gamma, beta, post_bias, W):
    nb = N // BN
    return pl.pallas_call(
        _stage_body,
        grid=(nb, KK),
        in_specs=[
            pl.BlockSpec((BN, C), lambda i, k: (i, 0)),
            pl.BlockSpec((BN, C), lambda i, k: (i, 0)),
            pl.BlockSpec((1, C), lambda i, k: (0, 0)),
            pl.BlockSpec((1, C), lambda i, k: (0, 0)),
            pl.BlockSpec((1, C), lambda i, k: (0, 0)),
            pl.BlockSpec((K, C, C), lambda i, k: (0, 0, 0)),
        ],
        out_specs=pl.BlockSpec((1, BN, C), lambda i, k: (k, i, 0)),
        out_shape=jax.ShapeDtypeStruct((KK, N, C), jnp.bfloat16),
        scratch_shapes=[pltpu.VMEM((BN, C), jnp.bfloat16)],
    )(x, resid, gamma.reshape(1, C), beta.reshape(1, C),
      post_bias.reshape(1, C), W.astype(jnp.bfloat16))


def _sc_body(table_hbm, idx_hbm, out_hbm, idx_v, buf0, buf1, ob,
             sem0a, sem0b, sem1a, sem1b):
    wid = lax.axis_index("s") * 2 + lax.axis_index("c")
    base = wid * RPW
    pltpu.sync_copy(idx_hbm.at[pl.ds(wid * RPW * KK, RPW * KK)], idx_v)

    def start(i, buf, sema, semb):
        o = i * CHR
        pltpu.async_copy(
            table_hbm.at[idx_v.at[pl.ds(o, CHH)]],
            buf.at[pl.ds(0, CHH)], sema)
        pltpu.async_copy(
            table_hbm.at[idx_v.at[pl.ds(o + CHH, CHH)]],
            buf.at[pl.ds(CHH, CHH)], semb)

    def wait(buf, sema, semb):
        pltpu.make_async_copy(
            table_hbm.at[idx_v.at[pl.ds(0, CHH)]],
            buf.at[pl.ds(0, CHH)], sema).wait()
        pltpu.make_async_copy(
            table_hbm.at[idx_v.at[pl.ds(0, CHH)]],
            buf.at[pl.ds(CHH, CHH)], semb).wait()

    def accum_and_emit(i, buf):
        mask = jnp.int32(-65536)

        def _lo(w):
            return jax.lax.bitcast_convert_type(w << 16, jnp.float32)

        def _hi(w):
            return jax.lax.bitcast_convert_type(w & mask, jnp.float32)

        @pl.loop(0, CH)
        def _(r):
            rb = r * KK
            for c in range(C // 32):
                w = buf[rb, pl.ds(c * 16, 16)]
                acc_lo, acc_hi = _lo(w), _hi(w)
                for j in range(1, KK):
                    w = buf[rb + j, pl.ds(c * 16, 16)]
                    acc_lo = acc_lo + _lo(w)
                    acc_hi = acc_hi + _hi(w)
                ob[r, pl.ds(c * 16, 16)] = acc_lo
                ob[r, pl.ds(C // 2 + c * 16, 16)] = acc_hi
        pltpu.sync_copy(ob, out_hbm.at[pl.ds(base + i * CH, CH)])

    start(0, buf0, sem0a, sem0b)

    @pl.loop(0, NCH, step=2)
    def _(ck):
        wait(buf0, sem0a, sem0b)
        start(ck + 1, buf1, sem1a, sem1b)
        accum_and_emit(ck, buf0)
        wait(buf1, sem1a, sem1b)

        @pl.when(ck + 2 < NCH)
        def _():
            start(ck + 2, buf0, sem0a, sem0b)

        accum_and_emit(ck + 1, buf1)


_gather_sum = pl.kernel(
    _sc_body,
    out_type=jax.ShapeDtypeStruct((NP, C), jnp.float32),
    mesh=plsc.VectorSubcoreMesh(core_axis_name="c", subcore_axis_name="s"),
    scratch_types=[
        pltpu.VMEM((RPW * KK,), jnp.int32),
        pltpu.VMEM((CHR, C // 2), jnp.int32),
        pltpu.VMEM((CHR, C // 2), jnp.int32),
        pltpu.VMEM((CH, C), jnp.float32),
        pltpu.SemaphoreType.DMA,
        pltpu.SemaphoreType.DMA,
        pltpu.SemaphoreType.DMA,
        pltpu.SemaphoreType.DMA,
    ],
)


def kernel(feats, nbr_idx, gamma1, beta1, W1, b1, W2, b2):
    nbr = nbr_idx.astype(jnp.int32)
    nbr_p = jnp.pad(nbr, ((0, NP - N), (0, 0)), mode="wrap")
    idxT = nbr_p.T + jnp.arange(K, dtype=jnp.int32)[:, None] * N
    rows = jnp.arange(NP, dtype=jnp.int32)
    ident = K * N + jnp.minimum(rows, N - 1)
    idx = (jnp.concatenate([idxT, ident[None]], axis=0)
           .reshape(KK, NW, RPW).transpose(1, 2, 0).reshape(-1))

    ones = jnp.ones((C,), jnp.float32)
    zeros = jnp.zeros((C,), jnp.float32)
    zrows = jnp.zeros((N, C), jnp.float32)

    def _pack(m):
        pair = jnp.stack([m[..., : C // 2], m[..., C // 2:]], axis=-1)
        return jax.lax.bitcast_convert_type(
            pair, jnp.int32).reshape(-1, C // 2)

    m1 = _pack(_stage_matmul(feats, zrows, gamma1, beta1, b1, W1))
    c1 = _gather_sum(m1, idx)[:N]

    m2 = _pack(_stage_matmul(c1, feats, ones, zeros, b2, W2))
    out = _gather_sum(m2, idx)[:N]
    return out


# speedup vs baseline: 2.0707x; 2.0707x over previous
import jax
import jax.numpy as jnp
from jax import lax
from jax.experimental import pallas as pl
from jax.experimental.pallas import tpu as pltpu
from jax.experimental.pallas import tpu_sc as plsc

N = 10000
C = 256
K = 27
KK = K + 1
EPS = 1e-6

NW = 32
NP = 10240
RPW = NP // NW
BN = 2000
CH = 8
NCH = RPW // CH
CHR = CH * KK
CHH = CHR // 2


def _pack_pairs(res):
    u = jax.lax.bitcast_convert_type(res, jnp.uint32) + jnp.uint32(0x8000)
    lo = u[:, : C // 2] >> 16
    hi = u[:, C // 2:] & jnp.uint32(0xFFFF0000)
    return jax.lax.bitcast_convert_type(lo | hi, jnp.int32)


def _stage_body(x_ref, f_ref, g_ref, b_ref, pb_ref, w_ref, out_ref, h_ref):
    k = pl.program_id(1)

    @pl.when(k == 0)
    def _():
        x = x_ref[...].astype(jnp.float32)
        mean = jnp.mean(x, axis=-1, keepdims=True)
        var = jnp.mean((x - mean) ** 2, axis=-1, keepdims=True)
        y = (x - mean) * lax.rsqrt(var + EPS)
        y = y * g_ref[0, :] + b_ref[0, :]
        h_ref[...] = (y * jax.nn.sigmoid(y)).astype(jnp.bfloat16)

    @pl.when(k < K)
    def _():
        out_ref[0] = _pack_pairs(jnp.dot(
            h_ref[...], w_ref[jnp.minimum(k, K - 1)],
            preferred_element_type=jnp.float32,
        ))

    @pl.when(k == K)
    def _():
        out_ref[0] = _pack_pairs(f_ref[...].astype(jnp.float32)
                                 + pb_ref[0, :])


def _stage_matmul(x, resid, gamma, beta, post_bias, W):
    nb = N // BN
    return pl.pallas_call(
        _stage_body,
        grid=(nb, KK),
        in_specs=[
            pl.BlockSpec((BN, C), lambda i, k: (i, 0)),
            pl.BlockSpec((BN, C), lambda i, k: (i, 0)),
            pl.BlockSpec((1, C), lambda i, k: (0, 0)),
            pl.BlockSpec((1, C), lambda i, k: (0, 0)),
            pl.BlockSpec((1, C), lambda i, k: (0, 0)),
            pl.BlockSpec((K, C, C), lambda i, k: (0, 0, 0)),
        ],
        out_specs=pl.BlockSpec((1, BN, C // 2), lambda i, k: (k, i, 0)),
        out_shape=jax.ShapeDtypeStruct((KK, N, C // 2), jnp.int32),
        scratch_shapes=[pltpu.VMEM((BN, C), jnp.bfloat16)],
    )(x, resid, gamma.reshape(1, C), beta.reshape(1, C),
      post_bias.reshape(1, C), W.astype(jnp.bfloat16))


def _sc_body(table_hbm, idx_hbm, out_hbm, idx_v, buf0, buf1, ob,
             sem0a, sem0b, sem1a, sem1b):
    wid = lax.axis_index("s") * 2 + lax.axis_index("c")
    base = wid * RPW
    pltpu.sync_copy(idx_hbm.at[pl.ds(wid * RPW * KK, RPW * KK)], idx_v)

    def start(i, buf, sema, semb):
        o = i * CHR
        pltpu.async_copy(
            table_hbm.at[idx_v.at[pl.ds(o, CHH)]],
            buf.at[pl.ds(0, CHH)], sema)
        pltpu.async_copy(
            table_hbm.at[idx_v.at[pl.ds(o + CHH, CHH)]],
            buf.at[pl.ds(CHH, CHH)], semb)

    def wait(buf, sema, semb):
        pltpu.make_async_copy(
            table_hbm.at[idx_v.at[pl.ds(0, CHH)]],
            buf.at[pl.ds(0, CHH)], sema).wait()
        pltpu.make_async_copy(
            table_hbm.at[idx_v.at[pl.ds(0, CHH)]],
            buf.at[pl.ds(CHH, CHH)], semb).wait()

    def accum_and_emit(i, buf):
        mask = jnp.int32(-65536)

        def _lo(w):
            return jax.lax.bitcast_convert_type(w << 16, jnp.float32)

        def _hi(w):
            return jax.lax.bitcast_convert_type(w & mask, jnp.float32)

        @pl.loop(0, CH)
        def _(r):
            rb = r * KK
            for c in range(C // 32):
                w = buf[rb, pl.ds(c * 16, 16)]
                acc_lo, acc_hi = _lo(w), _hi(w)
                for j in range(1, KK):
                    w = buf[rb + j, pl.ds(c * 16, 16)]
                    acc_lo = acc_lo + _lo(w)
                    acc_hi = acc_hi + _hi(w)
                ob[r, pl.ds(c * 16, 16)] = acc_lo
                ob[r, pl.ds(C // 2 + c * 16, 16)] = acc_hi
        pltpu.sync_copy(ob, out_hbm.at[pl.ds(base + i * CH, CH)])

    start(0, buf0, sem0a, sem0b)

    @pl.loop(0, NCH, step=2)
    def _(ck):
        wait(buf0, sem0a, sem0b)
        start(ck + 1, buf1, sem1a, sem1b)
        accum_and_emit(ck, buf0)
        wait(buf1, sem1a, sem1b)

        @pl.when(ck + 2 < NCH)
        def _():
            start(ck + 2, buf0, sem0a, sem0b)

        accum_and_emit(ck + 1, buf1)


_gather_sum = pl.kernel(
    _sc_body,
    out_type=jax.ShapeDtypeStruct((NP, C), jnp.float32),
    mesh=plsc.VectorSubcoreMesh(core_axis_name="c", subcore_axis_name="s"),
    scratch_types=[
        pltpu.VMEM((RPW * KK,), jnp.int32),
        pltpu.VMEM((CHR, C // 2), jnp.int32),
        pltpu.VMEM((CHR, C // 2), jnp.int32),
        pltpu.VMEM((CH, C), jnp.float32),
        pltpu.SemaphoreType.DMA,
        pltpu.SemaphoreType.DMA,
        pltpu.SemaphoreType.DMA,
        pltpu.SemaphoreType.DMA,
    ],
)


def kernel(feats, nbr_idx, gamma1, beta1, W1, b1, W2, b2):
    nbr = nbr_idx.astype(jnp.int32)
    nbr_p = jnp.pad(nbr, ((0, NP - N), (0, 0)), mode="wrap")
    idxT = nbr_p.T + jnp.arange(K, dtype=jnp.int32)[:, None] * N
    rows = jnp.arange(NP, dtype=jnp.int32)
    ident = K * N + jnp.minimum(rows, N - 1)
    idx = (jnp.concatenate([idxT, ident[None]], axis=0)
           .reshape(KK, NW, RPW).transpose(1, 2, 0).reshape(-1))

    ones = jnp.ones((C,), jnp.float32)
    zeros = jnp.zeros((C,), jnp.float32)
    zrows = jnp.zeros((N, C), jnp.float32)

    m1 = _stage_matmul(feats, zrows, gamma1, beta1, b1, W1).reshape(-1, C // 2)
    c1 = _gather_sum(m1, idx)[:N]

    m2 = _stage_matmul(c1, feats, ones, zeros, b2, W2).reshape(-1, C // 2)
    out = _gather_sum(m2, idx)[:N]
    return out


# speedup vs baseline: 2.4407x; 1.1787x over previous
import jax
import jax.numpy as jnp
from jax import lax
from jax.experimental import pallas as pl
from jax.experimental.pallas import tpu as pltpu
from jax.experimental.pallas import tpu_sc as plsc

N = 10000
C = 256
K = 27
KK = K + 1
EPS = 1e-6

NW = 32
NP = 10240
RPW = NP // NW
BN = 5000
CH = 8
NCH = RPW // CH
CHR = CH * KK
CHH = CHR // 2


def _pack_pairs(res):
    u = jax.lax.bitcast_convert_type(res, jnp.uint32) + jnp.uint32(0x8000)
    lo = u[:, : C // 2] >> 16
    hi = u[:, C // 2:] & jnp.uint32(0xFFFF0000)
    return jax.lax.bitcast_convert_type(lo | hi, jnp.int32)


def _stage_body(x_ref, f_ref, g_ref, b_ref, pb_ref, w_ref, out_ref, h_ref):
    k = pl.program_id(1)

    @pl.when(k == 0)
    def _():
        x = x_ref[...].astype(jnp.float32)
        mean = jnp.mean(x, axis=-1, keepdims=True)
        var = jnp.mean((x - mean) ** 2, axis=-1, keepdims=True)
        y = (x - mean) * lax.rsqrt(var + EPS)
        y = y * g_ref[0, :] + b_ref[0, :]
        h_ref[...] = (y * jax.nn.sigmoid(y)).astype(jnp.bfloat16)

    @pl.when(k < K)
    def _():
        out_ref[0] = _pack_pairs(jnp.dot(
            h_ref[...], w_ref[jnp.minimum(k, K - 1)],
            preferred_element_type=jnp.float32,
        ))

    @pl.when(k == K)
    def _():
        out_ref[0] = _pack_pairs(f_ref[...].astype(jnp.float32)
                                 + pb_ref[0, :])


def _stage_matmul(x, resid, gamma, beta, post_bias, W):
    nb = N // BN
    return pl.pallas_call(
        _stage_body,
        grid=(nb, KK),
        in_specs=[
            pl.BlockSpec((BN, C), lambda i, k: (i, 0)),
            pl.BlockSpec((BN, C), lambda i, k: (i, 0)),
            pl.BlockSpec((1, C), lambda i, k: (0, 0)),
            pl.BlockSpec((1, C), lambda i, k: (0, 0)),
            pl.BlockSpec((1, C), lambda i, k: (0, 0)),
            pl.BlockSpec((K, C, C), lambda i, k: (0, 0, 0)),
        ],
        out_specs=pl.BlockSpec((1, BN, C // 2), lambda i, k: (k, i, 0)),
        out_shape=jax.ShapeDtypeStruct((KK, N, C // 2), jnp.int32),
        scratch_shapes=[pltpu.VMEM((BN, C), jnp.bfloat16)],
    )(x, resid, gamma.reshape(1, C), beta.reshape(1, C),
      post_bias.reshape(1, C), W.astype(jnp.bfloat16))


def _sc_body(table_hbm, idx_hbm, out_hbm, idx_v, buf0, buf1, ob,
             sem0a, sem0b, sem1a, sem1b):
    wid = lax.axis_index("s") * 2 + lax.axis_index("c")
    base = wid * RPW
    pltpu.sync_copy(idx_hbm.at[pl.ds(wid * RPW * KK, RPW * KK)], idx_v)

    def start(i, buf, sema, semb):
        o = i * CHR
        pltpu.async_copy(
            table_hbm.at[idx_v.at[pl.ds(o, CHH)]],
            buf.at[pl.ds(0, CHH)], sema)
        pltpu.async_copy(
            table_hbm.at[idx_v.at[pl.ds(o + CHH, CHH)]],
            buf.at[pl.ds(CHH, CHH)], semb)

    def wait(buf, sema, semb):
        pltpu.make_async_copy(
            table_hbm.at[idx_v.at[pl.ds(0, CHH)]],
            buf.at[pl.ds(0, CHH)], sema).wait()
        pltpu.make_async_copy(
            table_hbm.at[idx_v.at[pl.ds(0, CHH)]],
            buf.at[pl.ds(CHH, CHH)], semb).wait()

    def accum_and_emit(i, buf):
        mask = jnp.int32(-65536)

        def _lo(w):
            return jax.lax.bitcast_convert_type(w << 16, jnp.float32)

        def _hi(w):
            return jax.lax.bitcast_convert_type(w & mask, jnp.float32)

        @pl.loop(0, CH)
        def _(r):
            rb = r * KK
            for c in range(C // 32):
                w = buf[rb, pl.ds(c * 16, 16)]
                acc_lo, acc_hi = _lo(w), _hi(w)
                for j in range(1, KK):
                    w = buf[rb + j, pl.ds(c * 16, 16)]
                    acc_lo = acc_lo + _lo(w)
                    acc_hi = acc_hi + _hi(w)
                ob[r, pl.ds(c * 16, 16)] = acc_lo
                ob[r, pl.ds(C // 2 + c * 16, 16)] = acc_hi
        pltpu.sync_copy(ob, out_hbm.at[pl.ds(base + i * CH, CH)])

    start(0, buf0, sem0a, sem0b)

    @pl.loop(0, NCH, step=2)
    def _(ck):
        wait(buf0, sem0a, sem0b)
        start(ck + 1, buf1, sem1a, sem1b)
        accum_and_emit(ck, buf0)
        wait(buf1, sem1a, sem1b)

        @pl.when(ck + 2 < NCH)
        def _():
            start(ck + 2, buf0, sem0a, sem0b)

        accum_and_emit(ck + 1, buf1)


_gather_sum = pl.kernel(
    _sc_body,
    out_type=jax.ShapeDtypeStruct((NP, C), jnp.float32),
    mesh=plsc.VectorSubcoreMesh(core_axis_name="c", subcore_axis_name="s"),
    scratch_types=[
        pltpu.VMEM((RPW * KK,), jnp.int32),
        pltpu.VMEM((CHR, C // 2), jnp.int32),
        pltpu.VMEM((CHR, C // 2), jnp.int32),
        pltpu.VMEM((CH, C), jnp.float32),
        pltpu.SemaphoreType.DMA,
        pltpu.SemaphoreType.DMA,
        pltpu.SemaphoreType.DMA,
        pltpu.SemaphoreType.DMA,
    ],
)


def kernel(feats, nbr_idx, gamma1, beta1, W1, b1, W2, b2):
    nbr = nbr_idx.astype(jnp.int32)
    nbr_p = jnp.pad(nbr, ((0, NP - N), (0, 0)), mode="wrap")
    idxT = nbr_p.T + jnp.arange(K, dtype=jnp.int32)[:, None] * N
    rows = jnp.arange(NP, dtype=jnp.int32)
    ident = K * N + jnp.minimum(rows, N - 1)
    idx = (jnp.concatenate([idxT, ident[None]], axis=0)
           .reshape(KK, NW, RPW).transpose(1, 2, 0).reshape(-1))

    ones = jnp.ones((C,), jnp.float32)
    zeros = jnp.zeros((C,), jnp.float32)
    zrows = jnp.zeros((N, C), jnp.float32)

    m1 = _stage_matmul(feats, zrows, gamma1, beta1, b1, W1).reshape(-1, C // 2)
    c1 = _gather_sum(m1, idx)[:N]

    m2 = _stage_matmul(c1, feats, ones, zeros, b2, W2).reshape(-1, C // 2)
    out = _gather_sum(m2, idx)[:N]
    return out
